# baseline (speedup 1.0000x reference)
import math

import jax
import jax.numpy as jnp
from jax import lax
from jax.experimental import pallas as pl
from jax.experimental.pallas import tpu as pltpu
import jax.experimental.pallas.tpu_sc as plsc

_N = 100000
_G = 1024
_T_MAX = 1000
_D = 128
_HALF = _D // 2

_NC = 2
_NS = 16
_NW = _NC * _NS
_BLK = 128
_NBLK = _N // _BLK
_TAIL = _N - _NBLK * _BLK
_EXTRA = _NBLK - 24 * _NW
_MAXB = 25
_CH = _MAXB * _BLK
_TAIL_OFF = _NBLK * _BLK

_SC_PARAMS = pltpu.CompilerParams(needs_layout_passes=False,
                                  use_tc_tiling_on_sc=False)


def _tables_body(t_ref, alpha_ref, sigma_ref, emb_ref):
    tf = t_ref[...].astype(jnp.float32)
    ang = tf * (0.5 * math.pi / _T_MAX)
    alpha_ref[...] = jnp.cos(ang)
    sigma_ref[...] = jnp.sin(ang)
    j = lax.broadcasted_iota(jnp.int32, (_G, _HALF), 1).astype(jnp.float32)
    freqs = jnp.exp(j * (-math.log(10000.0) / _HALF))
    args = tf * freqs
    emb_ref[:, :_HALF] = jnp.sin(args)
    emb_ref[:, _HALF:] = jnp.cos(args)


def _tables(t2):
    return pl.pallas_call(
        _tables_body,
        out_shape=(
            jax.ShapeDtypeStruct((_G, 1), jnp.float32),
            jax.ShapeDtypeStruct((_G, 1), jnp.float32),
            jax.ShapeDtypeStruct((_G, _D), jnp.float32),
        ),
    )(t2)


def _combine_body(p_ref, a_ref, s_ref, t8_ref):
    p = p_ref[0] + p_ref[1]
    cnt = jnp.maximum(p[:, 6:7], 1.0)
    t8_ref[:, 0:6] = p[:, 0:6] / cnt
    t8_ref[:, 6:7] = a_ref[...]
    t8_ref[:, 7:8] = s_ref[...]


def _combine(partials, alpha, sigma):
    return pl.pallas_call(
        _combine_body,
        out_shape=jax.ShapeDtypeStruct((_G, 8), jnp.float32),
    )(partials, alpha, sigma)


def _worker_id():
    return lax.axis_index("s") * _NC + lax.axis_index("c")


def _chunk_of(wid):
    nb = 24 + jnp.where(wid < _EXTRA, 1, 0)
    base_blk = 24 * wid + jnp.minimum(wid, _EXTRA)
    return nb, base_blk * _BLK


def _iota16():
    return lax.iota(jnp.int32, 16)


def _mesh():
    return plsc.VectorSubcoreMesh(core_axis_name="c", subcore_axis_name="s")


def _full16(v):
    return jnp.full((16,), v, jnp.int32)


def _copy_chunk_in(src2d, dst, rb, nb):
    @pl.when(nb == _MAXB)
    def _a():
        pltpu.sync_copy(src2d.at[pl.ds(rb, _MAXB * _BLK), :], dst)

    @pl.when(nb == _MAXB - 1)
    def _b():
        pltpu.sync_copy(src2d.at[pl.ds(rb, (_MAXB - 1) * _BLK), :],
                        dst.at[pl.ds(0, (_MAXB - 1) * _BLK), :])


def _copy_chunk_out(src, dst2d, rb, nb):
    @pl.when(nb == _MAXB)
    def _a():
        pltpu.sync_copy(src, dst2d.at[pl.ds(rb, _MAXB * _BLK), :])

    @pl.when(nb == _MAXB - 1)
    def _b():
        pltpu.sync_copy(src.at[pl.ds(0, (_MAXB - 1) * _BLK), :],
                        dst2d.at[pl.ds(rb, (_MAXB - 1) * _BLK), :])


def _build_vals(k0, n16, ichunk, pchunk, echunk, vals):
    iota = _iota16()
    ones = jnp.ones((16,), jnp.float32)
    for k in range(n16):
        lrow = (k0 + k) * 16 + iota
        vrow = k * 16 + iota
        for base, buf in ((0, pchunk), (3, echunk)):
            for c in range(3):
                v = plsc.load_gather(buf, [lrow, _full16(c)])
                plsc.store_scatter(vals, [vrow, _full16(base + c)], v)
        plsc.store_scatter(vals, [vrow, _full16(6)], ones)


def _zero_col7(n16, vals):
    iota = _iota16()
    zeros = jnp.zeros((16,), jnp.float32)
    for k in range(n16):
        plsc.store_scatter(vals, [k * 16 + iota, _full16(7)], zeros)


def _sums_body(pos, eps, batch, out, acc, pchunk, echunk,
               idx0, idx1, vals0, vals1, ptail, etail, idx_t, vals_t,
               zbuf, obuf, sem0, sem1):
    c = lax.axis_index("c")
    s = lax.axis_index("s")
    wid = _worker_id()

    iota = _iota16()
    zeros = jnp.zeros((16,), jnp.float32)
    for i in range(32):
        plsc.store_scatter(zbuf, [i * 2 + iota // 8, iota % 8], zeros)
    pltpu.sync_copy(zbuf, acc.at[pl.ds(s * 64, 64), :])
    plsc.subcore_barrier()

    nb, rb = _chunk_of(wid)
    _copy_chunk_in(pos, pchunk, rb, nb)
    _copy_chunk_in(eps, echunk, rb, nb)
    _zero_col7(8, vals0)
    _zero_col7(8, vals1)

    def blk(j, carry):
        pltpu.sync_copy(batch.at[pl.ds(rb + j * _BLK, _BLK)], idx0)
        iota = _iota16()
        ones = jnp.ones((16,), jnp.float32)
        for k in range(8):
            lrow = j * _BLK + k * 16 + iota
            vrow = k * 16 + iota
            for base, buf in ((0, pchunk), (3, echunk)):
                for c in range(3):
                    v = plsc.load_gather(buf, [lrow, _full16(c)])
                    plsc.store_scatter(vals0, [vrow, _full16(base + c)], v)
            plsc.store_scatter(vals0, [vrow, _full16(6)], ones)
        pltpu.sync_copy(vals0, acc.at[idx0], add=True)
        return carry

    lax.fori_loop(0, nb, blk, 0)

    @pl.when(wid == _NW - 1)
    def _tail():
        r0 = _TAIL_OFF
        pltpu.sync_copy(batch.at[pl.ds(r0, _TAIL)], idx_t)
        pltpu.sync_copy(pos.at[pl.ds(r0, _TAIL), :], ptail)
        pltpu.sync_copy(eps.at[pl.ds(r0, _TAIL), :], etail)
        _zero_col7(2, vals_t)
        _build_vals(0, 2, idx_t, ptail, etail, vals_t)
        pltpu.sync_copy(vals_t, acc.at[idx_t], add=True)

    plsc.subcore_barrier()
    pltpu.sync_copy(acc.at[pl.ds(s * 64, 64), :], obuf)
    pltpu.sync_copy(obuf, out.at[c, pl.ds(s * 64, 64), :])


def _sc_sums(pos, eps, batch):
    f = pl.kernel(
        _sums_body,
        out_type=jax.ShapeDtypeStruct((_NC, _G, 8), jnp.float32),
        mesh=_mesh(),
        compiler_params=_SC_PARAMS,
        scratch_types=[
            pltpu.VMEM_SHARED((_G, 8), jnp.float32),
            pltpu.VMEM((_CH, 3), jnp.float32),
            pltpu.VMEM((_CH, 3), jnp.float32),
            pltpu.VMEM((_BLK,), jnp.int32),
            pltpu.VMEM((_BLK,), jnp.int32),
            pltpu.VMEM((_BLK, 8), jnp.float32),
            pltpu.VMEM((_BLK, 8), jnp.float32),
            pltpu.VMEM((_TAIL, 3), jnp.float32),
            pltpu.VMEM((_TAIL, 3), jnp.float32),
            pltpu.VMEM((_TAIL,), jnp.int32),
            pltpu.VMEM((_TAIL, 8), jnp.float32),
            pltpu.VMEM((64, 8), jnp.float32),
            pltpu.VMEM((64, 8), jnp.float32),
            pltpu.SemaphoreType.DMA,
            pltpu.SemaphoreType.DMA,
        ],
    )
    return f(pos, eps, batch)


def _center_rows(n16, k0, ichunk_vals, pchunk, echunk, t8v, pochunk):
    iota = _iota16()
    for k in range(n16):
        lrow = (k0 + k) * 16 + iota
        g = plsc.load_gather(ichunk_vals, [lrow])
        a = plsc.load_gather(t8v, [g, _full16(6)])
        sg = plsc.load_gather(t8v, [g, _full16(7)])
        for c in range(3):
            px = plsc.load_gather(pchunk, [lrow, _full16(c)])
            ex = plsc.load_gather(echunk, [lrow, _full16(c)])
            mp = plsc.load_gather(t8v, [g, _full16(c)])
            me = plsc.load_gather(t8v, [g, _full16(3 + c)])
            x = px - mp
            e = ex - me
            plsc.store_scatter(pochunk, [lrow, _full16(3 + c)], e)
            plsc.store_scatter(pochunk, [lrow, _full16(c)], a * x + sg * e)


def _center_body(pos, eps, batch, table8, po, t8v, ichunk,
                 pchunk, echunk, pochunk,
                 ptail, etail, idx_t, potail):
    wid = _worker_id()
    nb, rb = _chunk_of(wid)
    pltpu.sync_copy(table8, t8v)
    pltpu.sync_copy(batch.at[pl.ds(rb, _CH - _BLK)], ichunk.at[pl.ds(0, _CH - _BLK)])

    @pl.when(nb == _MAXB)
    def _i25():
        pltpu.sync_copy(batch.at[pl.ds(rb + _CH - _BLK, _BLK)],
                        ichunk.at[pl.ds(_CH - _BLK, _BLK)])

    _copy_chunk_in(pos, pchunk, rb, nb)
    _copy_chunk_in(eps, echunk, rb, nb)
    for j in range(_MAXB):
        @pl.when(j < nb)
        def _do():
            _center_rows(8, j * 8, ichunk, pchunk, echunk, t8v, pochunk)
    _copy_chunk_out(pochunk, po, rb, nb)

    @pl.when(wid == _NW - 1)
    def _tail():
        r0 = _TAIL_OFF
        pltpu.sync_copy(batch.at[pl.ds(r0, _TAIL)], idx_t)
        pltpu.sync_copy(pos.at[pl.ds(r0, _TAIL), :], ptail)
        pltpu.sync_copy(eps.at[pl.ds(r0, _TAIL), :], etail)
        _center_rows(2, 0, idx_t, ptail, etail, t8v, potail)
        pltpu.sync_copy(potail, po.at[pl.ds(r0, _TAIL), :])


def _sc_center(pos, eps, batch, table8):
    f = pl.kernel(
        _center_body,
        out_type=jax.ShapeDtypeStruct((_N, 8), jnp.float32),
        mesh=_mesh(),
        compiler_params=_SC_PARAMS,
        scratch_types=[
            pltpu.VMEM((_G, 8), jnp.float32),
            pltpu.VMEM((_CH,), jnp.int32),
            pltpu.VMEM((_CH, 3), jnp.float32),
            pltpu.VMEM((_CH, 3), jnp.float32),
            pltpu.VMEM((_CH, 8), jnp.float32),
            pltpu.VMEM((_TAIL, 3), jnp.float32),
            pltpu.VMEM((_TAIL, 3), jnp.float32),
            pltpu.VMEM((_TAIL,), jnp.int32),
            pltpu.VMEM((_TAIL, 8), jnp.float32),
        ],
    )
    return f(pos, eps, batch, table8)


_NBUF = 3


def _cond_body(batch, emb, cond, idxs, rows, semsG, semsW, idx_t, rows_t, semT):
    wid = _worker_id()
    nb, rb = _chunk_of(wid)

    def _gather(j, slot):
        return pltpu.make_async_copy(emb.at[idxs[slot]], rows[slot],
                                     semsG[slot])

    def _write(j, slot):
        return pltpu.make_async_copy(
            rows[slot], cond.at[pl.ds(rb + j * _BLK, _BLK), :], semsW[slot])

    for j in range(_MAXB + 2):
        slot = j % _NBUF
        if j < _MAXB:
            @pl.when(j < nb)
            def _issue():
                if j >= _NBUF:
                    _write(j - _NBUF, slot).wait()
                pltpu.sync_copy(batch.at[pl.ds(rb + j * _BLK, _BLK)],
                                idxs[slot])
                pltpu.async_copy(emb.at[idxs[slot]], rows[slot], semsG[slot])

        if j >= 2:
            jj = j - 2
            wslot = jj % _NBUF

            @pl.when(jj < nb)
            def _fin():
                _gather(jj, wslot).wait()
                pltpu.async_copy(rows[wslot],
                                 cond.at[pl.ds(rb + jj * _BLK, _BLK), :],
                                 semsW[wslot])

    for j in range(_MAXB - _NBUF - 1, _MAXB):
        slot = j % _NBUF

        @pl.when(jnp.logical_and(j < nb, j + _NBUF >= nb))
        def _drain():
            _write(j, slot).wait()

    @pl.when(wid == _NW - 1)
    def _tail():
        r0 = _TAIL_OFF
        pltpu.sync_copy(batch.at[pl.ds(r0, _TAIL)], idx_t)
        pltpu.async_copy(emb.at[idx_t], rows_t, semT).wait()
        pltpu.sync_copy(rows_t, cond.at[pl.ds(r0, _TAIL), :])


def _cond_body_wrap(batch, emb, cond,
                    idx0, idx1, idx2, rows0, rows1, rows2,
                    semG0, semG1, semG2, semW0, semW1, semW2,
                    idx_t, rows_t, semT):
    _cond_body(batch, emb, cond, (idx0, idx1, idx2), (rows0, rows1, rows2),
               (semG0, semG1, semG2), (semW0, semW1, semW2),
               idx_t, rows_t, semT)


def _sc_cond(batch, emb):
    f = pl.kernel(
        _cond_body_wrap,
        out_type=jax.ShapeDtypeStruct((_N, _D), jnp.float32),
        mesh=_mesh(),
        compiler_params=_SC_PARAMS,
        scratch_types=(
            [pltpu.VMEM((_BLK,), jnp.int32)] * 3
            + [pltpu.VMEM((_BLK, _D), jnp.float32)] * 3
            + [pltpu.SemaphoreType.DMA] * 6
            + [pltpu.VMEM((_TAIL,), jnp.int32),
               pltpu.VMEM((_TAIL, _D), jnp.float32),
               pltpu.SemaphoreType.DMA]
        ),
    )
    return f(batch, emb)


def kernel(pos, eps, batch, t):
    t2 = t.reshape(_G, 1)
    alpha, sigma, emb = _tables(t2)
    partials = _sc_sums(pos, eps, batch)
    table8 = _combine(partials, alpha, sigma)
    po8 = _sc_center(pos, eps, batch, table8)
    emb2, po8 = lax.optimization_barrier((emb, po8))
    cond = _sc_cond(batch, emb2)
    return (po8[:, 0:3], po8[:, 3:6], cond, alpha, sigma)


# speedup vs baseline: 1.1091x; 1.1091x over previous
import math

import jax
import jax.numpy as jnp
from jax import lax
from jax.experimental import pallas as pl
from jax.experimental.pallas import tpu as pltpu
import jax.experimental.pallas.tpu_sc as plsc

_N = 100000
_G = 1024
_T_MAX = 1000
_D = 128
_HALF = _D // 2

_NC = 2
_NS = 16
_NW = _NC * _NS
_BLK = 128
_NBLK = _N // _BLK
_TAIL = _N - _NBLK * _BLK
_EXTRA = _NBLK - 24 * _NW
_MAXB = 25
_CH = _MAXB * _BLK
_TAIL_OFF = _NBLK * _BLK

_SC_PARAMS = pltpu.CompilerParams(needs_layout_passes=False,
                                  use_tc_tiling_on_sc=False)


def _tables_body(t_ref, alpha_ref, sigma_ref, emb_ref):
    tf = t_ref[...].astype(jnp.float32)
    ang = tf * (0.5 * math.pi / _T_MAX)
    alpha_ref[...] = jnp.cos(ang)
    sigma_ref[...] = jnp.sin(ang)
    j = lax.broadcasted_iota(jnp.int32, (_G, _HALF), 1).astype(jnp.float32)
    freqs = jnp.exp(j * (-math.log(10000.0) / _HALF))
    args = tf * freqs
    emb_ref[:, :_HALF] = jnp.sin(args)
    emb_ref[:, _HALF:] = jnp.cos(args)


def _tables(t2):
    return pl.pallas_call(
        _tables_body,
        out_shape=(
            jax.ShapeDtypeStruct((_G, 1), jnp.float32),
            jax.ShapeDtypeStruct((_G, 1), jnp.float32),
            jax.ShapeDtypeStruct((_G, _D), jnp.float32),
        ),
    )(t2)


def _combine_body(p_ref, a_ref, s_ref, t8_ref):
    p = p_ref[0] + p_ref[1]
    cnt = jnp.maximum(p[:, 6:7], 1.0)
    t8_ref[:, 0:6] = p[:, 0:6] / cnt
    t8_ref[:, 6:7] = a_ref[...]
    t8_ref[:, 7:8] = s_ref[...]


def _combine(partials, alpha, sigma):
    return pl.pallas_call(
        _combine_body,
        out_shape=jax.ShapeDtypeStruct((_G, 8), jnp.float32),
    )(partials, alpha, sigma)


def _worker_id():
    return lax.axis_index("s") * _NC + lax.axis_index("c")


def _chunk_of(wid):
    nb = 24 + jnp.where(wid < _EXTRA, 1, 0)
    base_blk = 24 * wid + jnp.minimum(wid, _EXTRA)
    return nb, base_blk * _BLK


def _iota16():
    return lax.iota(jnp.int32, 16)


def _mesh():
    return plsc.VectorSubcoreMesh(core_axis_name="c", subcore_axis_name="s")


def _full16(v):
    return jnp.full((16,), v, jnp.int32)


def _copy_chunk_in(src2d, dst, rb, nb):
    @pl.when(nb == _MAXB)
    def _a():
        pltpu.sync_copy(src2d.at[pl.ds(rb, _MAXB * _BLK), :], dst)

    @pl.when(nb == _MAXB - 1)
    def _b():
        pltpu.sync_copy(src2d.at[pl.ds(rb, (_MAXB - 1) * _BLK), :],
                        dst.at[pl.ds(0, (_MAXB - 1) * _BLK), :])


def _copy_chunk_out(src, dst2d, rb, nb):
    @pl.when(nb == _MAXB)
    def _a():
        pltpu.sync_copy(src, dst2d.at[pl.ds(rb, _MAXB * _BLK), :])

    @pl.when(nb == _MAXB - 1)
    def _b():
        pltpu.sync_copy(src.at[pl.ds(0, (_MAXB - 1) * _BLK), :],
                        dst2d.at[pl.ds(rb, (_MAXB - 1) * _BLK), :])


def _build_vals(k0, n16, ichunk, pchunk, echunk, vals):
    iota = _iota16()
    ones = jnp.ones((16,), jnp.float32)
    for k in range(n16):
        lrow = (k0 + k) * 16 + iota
        vrow = k * 16 + iota
        for base, buf in ((0, pchunk), (3, echunk)):
            for c in range(3):
                v = plsc.load_gather(buf, [lrow, _full16(c)])
                plsc.store_scatter(vals, [vrow, _full16(base + c)], v)
        plsc.store_scatter(vals, [vrow, _full16(6)], ones)


def _zero_col7(n16, vals):
    iota = _iota16()
    zeros = jnp.zeros((16,), jnp.float32)
    for k in range(n16):
        plsc.store_scatter(vals, [k * 16 + iota, _full16(7)], zeros)


def _sums_body(pos, eps, batch, out, acc, pchunk, echunk,
               idx0, idx1, vals0, vals1, ptail, etail, idx_t, vals_t,
               zbuf, obuf, sem0, sem1):
    c = lax.axis_index("c")
    s = lax.axis_index("s")
    wid = _worker_id()

    iota = _iota16()
    zeros = jnp.zeros((16,), jnp.float32)
    for i in range(32):
        plsc.store_scatter(zbuf, [i * 2 + iota // 8, iota % 8], zeros)
    pltpu.sync_copy(zbuf, acc.at[pl.ds(s * 64, 64), :])
    plsc.subcore_barrier()

    nb, rb = _chunk_of(wid)
    _copy_chunk_in(pos, pchunk, rb, nb)
    _copy_chunk_in(eps, echunk, rb, nb)
    _zero_col7(8, vals0)
    _zero_col7(8, vals1)

    def blk(j, carry):
        pltpu.sync_copy(batch.at[pl.ds(rb + j * _BLK, _BLK)], idx0)
        iota = _iota16()
        ones = jnp.ones((16,), jnp.float32)
        for k in range(8):
            lrow = j * _BLK + k * 16 + iota
            vrow = k * 16 + iota
            for base, buf in ((0, pchunk), (3, echunk)):
                for c in range(3):
                    v = plsc.load_gather(buf, [lrow, _full16(c)])
                    plsc.store_scatter(vals0, [vrow, _full16(base + c)], v)
            plsc.store_scatter(vals0, [vrow, _full16(6)], ones)
        pltpu.sync_copy(vals0, acc.at[idx0], add=True)
        return carry

    lax.fori_loop(0, nb, blk, 0)

    @pl.when(wid == _NW - 1)
    def _tail():
        r0 = _TAIL_OFF
        pltpu.sync_copy(batch.at[pl.ds(r0, _TAIL)], idx_t)
        pltpu.sync_copy(pos.at[pl.ds(r0, _TAIL), :], ptail)
        pltpu.sync_copy(eps.at[pl.ds(r0, _TAIL), :], etail)
        _zero_col7(2, vals_t)
        _build_vals(0, 2, idx_t, ptail, etail, vals_t)
        pltpu.sync_copy(vals_t, acc.at[idx_t], add=True)

    plsc.subcore_barrier()
    pltpu.sync_copy(acc.at[pl.ds(s * 64, 64), :], obuf)
    pltpu.sync_copy(obuf, out.at[c, pl.ds(s * 64, 64), :])


def _sc_sums(pos, eps, batch):
    f = pl.kernel(
        _sums_body,
        out_type=jax.ShapeDtypeStruct((_NC, _G, 8), jnp.float32),
        mesh=_mesh(),
        compiler_params=_SC_PARAMS,
        scratch_types=[
            pltpu.VMEM_SHARED((_G, 8), jnp.float32),
            pltpu.VMEM((_CH, 3), jnp.float32),
            pltpu.VMEM((_CH, 3), jnp.float32),
            pltpu.VMEM((_BLK,), jnp.int32),
            pltpu.VMEM((_BLK,), jnp.int32),
            pltpu.VMEM((_BLK, 8), jnp.float32),
            pltpu.VMEM((_BLK, 8), jnp.float32),
            pltpu.VMEM((_TAIL, 3), jnp.float32),
            pltpu.VMEM((_TAIL, 3), jnp.float32),
            pltpu.VMEM((_TAIL,), jnp.int32),
            pltpu.VMEM((_TAIL, 8), jnp.float32),
            pltpu.VMEM((64, 8), jnp.float32),
            pltpu.VMEM((64, 8), jnp.float32),
            pltpu.SemaphoreType.DMA,
            pltpu.SemaphoreType.DMA,
        ],
    )
    return f(pos, eps, batch)


def _center_rows(n16, k0, ichunk_vals, pchunk, echunk, t8v, pochunk, eochunk):
    iota = _iota16()
    for k in range(n16):
        lrow = (k0 + k) * 16 + iota
        g = plsc.load_gather(ichunk_vals, [lrow])
        a = plsc.load_gather(t8v, [g, _full16(6)])
        sg = plsc.load_gather(t8v, [g, _full16(7)])
        for c in range(3):
            px = plsc.load_gather(pchunk, [lrow, _full16(c)])
            ex = plsc.load_gather(echunk, [lrow, _full16(c)])
            mp = plsc.load_gather(t8v, [g, _full16(c)])
            me = plsc.load_gather(t8v, [g, _full16(3 + c)])
            x = px - mp
            e = ex - me
            plsc.store_scatter(eochunk, [lrow, _full16(c)], e)
            plsc.store_scatter(pochunk, [lrow, _full16(c)], a * x + sg * e)


def _center_body(pos, eps, batch, table8, po, eo, t8v, ichunk,
                 pchunk, echunk, pochunk, eochunk,
                 ptail, etail, idx_t, potail, eotail):
    wid = _worker_id()
    nb, rb = _chunk_of(wid)
    pltpu.sync_copy(table8, t8v)
    pltpu.sync_copy(batch.at[pl.ds(rb, _CH - _BLK)], ichunk.at[pl.ds(0, _CH - _BLK)])

    @pl.when(nb == _MAXB)
    def _i25():
        pltpu.sync_copy(batch.at[pl.ds(rb + _CH - _BLK, _BLK)],
                        ichunk.at[pl.ds(_CH - _BLK, _BLK)])

    _copy_chunk_in(pos, pchunk, rb, nb)
    _copy_chunk_in(eps, echunk, rb, nb)
    for j in range(_MAXB):
        @pl.when(j < nb)
        def _do():
            _center_rows(8, j * 8, ichunk, pchunk, echunk, t8v,
                         pochunk, eochunk)
    _copy_chunk_out(pochunk, po, rb, nb)
    _copy_chunk_out(eochunk, eo, rb, nb)

    @pl.when(wid == _NW - 1)
    def _tail():
        r0 = _TAIL_OFF
        pltpu.sync_copy(batch.at[pl.ds(r0, _TAIL)], idx_t)
        pltpu.sync_copy(pos.at[pl.ds(r0, _TAIL), :], ptail)
        pltpu.sync_copy(eps.at[pl.ds(r0, _TAIL), :], etail)
        _center_rows(2, 0, idx_t, ptail, etail, t8v, potail, eotail)
        pltpu.sync_copy(potail, po.at[pl.ds(r0, _TAIL), :])
        pltpu.sync_copy(eotail, eo.at[pl.ds(r0, _TAIL), :])


def _sc_center(pos, eps, batch, table8):
    f = pl.kernel(
        _center_body,
        out_type=(
            jax.ShapeDtypeStruct((_N, 3), jnp.float32),
            jax.ShapeDtypeStruct((_N, 3), jnp.float32),
        ),
        mesh=_mesh(),
        compiler_params=_SC_PARAMS,
        scratch_types=[
            pltpu.VMEM((_G, 8), jnp.float32),
            pltpu.VMEM((_CH,), jnp.int32),
            pltpu.VMEM((_CH, 3), jnp.float32),
            pltpu.VMEM((_CH, 3), jnp.float32),
            pltpu.VMEM((_CH, 3), jnp.float32),
            pltpu.VMEM((_CH, 3), jnp.float32),
            pltpu.VMEM((_TAIL, 3), jnp.float32),
            pltpu.VMEM((_TAIL, 3), jnp.float32),
            pltpu.VMEM((_TAIL,), jnp.int32),
            pltpu.VMEM((_TAIL, 3), jnp.float32),
            pltpu.VMEM((_TAIL, 3), jnp.float32),
        ],
    )
    return f(pos, eps, batch, table8)


_NBUF = 3


def _cond_body(batch, emb, cond, idxs, rows, semsG, semsW, idx_t, rows_t, semT):
    wid = _worker_id()
    nb, rb = _chunk_of(wid)

    def _gather(j, slot):
        return pltpu.make_async_copy(emb.at[idxs[slot]], rows[slot],
                                     semsG[slot])

    def _write(j, slot):
        return pltpu.make_async_copy(
            rows[slot], cond.at[pl.ds(rb + j * _BLK, _BLK), :], semsW[slot])

    for j in range(_MAXB + 2):
        slot = j % _NBUF
        if j < _MAXB:
            @pl.when(j < nb)
            def _issue():
                if j >= _NBUF:
                    _write(j - _NBUF, slot).wait()
                pltpu.sync_copy(batch.at[pl.ds(rb + j * _BLK, _BLK)],
                                idxs[slot])
                pltpu.async_copy(emb.at[idxs[slot]], rows[slot], semsG[slot])

        if j >= 2:
            jj = j - 2
            wslot = jj % _NBUF

            @pl.when(jj < nb)
            def _fin():
                _gather(jj, wslot).wait()
                pltpu.async_copy(rows[wslot],
                                 cond.at[pl.ds(rb + jj * _BLK, _BLK), :],
                                 semsW[wslot])

    for j in range(_MAXB - _NBUF - 1, _MAXB):
        slot = j % _NBUF

        @pl.when(jnp.logical_and(j < nb, j + _NBUF >= nb))
        def _drain():
            _write(j, slot).wait()

    @pl.when(wid == _NW - 1)
    def _tail():
        r0 = _TAIL_OFF
        pltpu.sync_copy(batch.at[pl.ds(r0, _TAIL)], idx_t)
        pltpu.async_copy(emb.at[idx_t], rows_t, semT).wait()
        pltpu.sync_copy(rows_t, cond.at[pl.ds(r0, _TAIL), :])


def _cond_body_wrap(batch, emb, cond,
                    idx0, idx1, idx2, rows0, rows1, rows2,
                    semG0, semG1, semG2, semW0, semW1, semW2,
                    idx_t, rows_t, semT):
    _cond_body(batch, emb, cond, (idx0, idx1, idx2), (rows0, rows1, rows2),
               (semG0, semG1, semG2), (semW0, semW1, semW2),
               idx_t, rows_t, semT)


def _sc_cond(batch, emb):
    f = pl.kernel(
        _cond_body_wrap,
        out_type=jax.ShapeDtypeStruct((_N, _D), jnp.float32),
        mesh=_mesh(),
        compiler_params=_SC_PARAMS,
        scratch_types=(
            [pltpu.VMEM((_BLK,), jnp.int32)] * 3
            + [pltpu.VMEM((_BLK, _D), jnp.float32)] * 3
            + [pltpu.SemaphoreType.DMA] * 6
            + [pltpu.VMEM((_TAIL,), jnp.int32),
               pltpu.VMEM((_TAIL, _D), jnp.float32),
               pltpu.SemaphoreType.DMA]
        ),
    )
    return f(batch, emb)


def kernel(pos, eps, batch, t):
    t2 = t.reshape(_G, 1)
    alpha, sigma, emb = _tables(t2)
    partials = _sc_sums(pos, eps, batch)
    table8 = _combine(partials, alpha, sigma)
    pos_out, eps_c = _sc_center(pos, eps, batch, table8)
    emb2, pos_out = lax.optimization_barrier((emb, pos_out))
    cond = _sc_cond(batch, emb2)
    return (pos_out, eps_c, cond, alpha, sigma)
